# baseline (speedup 1.0000x reference)
import functools

import jax
import jax.numpy as jnp
from jax import lax
from jax.experimental import pallas as pl
from jax.experimental.pallas import tpu as pltpu
from jax.experimental.pallas import tpu_sc as plsc

N_NODES = 10000
N_EDGES = 160000
LAT = 256
NOUT = 128
CH = 128
NCHUNK = N_EDGES // CH
HALF = N_NODES // 2
ACC_ROWS = HALF + 8


def _mesh():
    return plsc.VectorSubcoreMesh(core_axis_name="c", subcore_axis_name="s")


def _gather2_body(a_hbm, b_hbm, s_hbm, r_hbm, ag_hbm, bg_hbm,
                  sidx, ridx, abuf, bbuf, sem):
    cid = lax.axis_index("c")
    sid = lax.axis_index("s")
    wid = sid * 2 + cid
    nch = (NCHUNK - wid + 31) // 32

    def chunk(i, _):
        base = pl.multiple_of((wid + i * 32) * CH, 8)
        pltpu.sync_copy(s_hbm.at[pl.ds(base, CH)], sidx)
        pltpu.sync_copy(r_hbm.at[pl.ds(base, CH)], ridx)
        ca = pltpu.async_copy(a_hbm.at[sidx], abuf, sem)
        cb = pltpu.async_copy(b_hbm.at[ridx], bbuf, sem)
        ca.wait()
        cb.wait()
        pltpu.sync_copy(abuf, ag_hbm.at[pl.ds(base, CH)])
        pltpu.sync_copy(bbuf, bg_hbm.at[pl.ds(base, CH)])
        return 0

    lax.fori_loop(0, nch, chunk, 0)


def _gather2(a, b, senders, receivers):
    f = functools.partial(
        pl.kernel, _gather2_body,
        out_type=(jax.ShapeDtypeStruct((N_EDGES, LAT), jnp.float32),
                  jax.ShapeDtypeStruct((N_EDGES, LAT), jnp.float32)),
        mesh=_mesh(),
        scratch_types=[
            pltpu.VMEM((CH,), jnp.int32),
            pltpu.VMEM((CH,), jnp.int32),
            pltpu.VMEM((CH, LAT), jnp.float32),
            pltpu.VMEM((CH, LAT), jnp.float32),
            pltpu.SemaphoreType.DMA,
        ],
    )()
    return f(a, b, senders, receivers)


NT = 312
NTL = 328
ACC_R = 336


def _segsum_body(h_hbm, perm_hbm, nid_hbm, tb_hbm, out_hbm,
                 pidx, nbuf, tbuf, hbuf, acc, sem):
    cid = lax.axis_index("c")
    sid = lax.axis_index("s")
    wid = sid * 2 + cid
    base_node = NT * wid
    nn = jnp.where(wid == 31, NTL, NT)

    def zrow(r, _):
        for k in range(LAT // 16):
            acc[r, pl.ds(k * 16, 16)] = jnp.zeros((16,), jnp.float32)
        return 0

    lax.fori_loop(0, ACC_R, zrow, 0)

    pltpu.sync_copy(tb_hbm.at[pl.ds(pl.multiple_of(8 * wid, 8), 16)], tbuf)
    tv = tbuf[...]
    lo = tv[0]
    hi = tv[1]
    ca0 = (lo // CH) * CH
    nch = (hi - ca0 + CH - 1) // CH

    def chunk(i, _):
        ca = pl.multiple_of(ca0 + i * CH, 8)
        pltpu.sync_copy(perm_hbm.at[pl.ds(ca, CH)], pidx)
        pltpu.sync_copy(nid_hbm.at[pl.ds(ca, CH)], nbuf)
        pltpu.async_copy(h_hbm.at[pidx], hbuf, sem).wait()
        for g in range(CH // 16):
            nv = nbuf[pl.ds(g * 16, 16)]
            for j in range(16):
                r = g * 16 + j
                li = nv[j] - base_node
                ok = jnp.logical_and(li >= 0, li < nn)
                li = jnp.where(ok, li, NTL)
                for k in range(LAT // 16):
                    sl = pl.ds(k * 16, 16)
                    plsc.addupdate(acc.at[li, sl], hbuf[r, sl])
        return 0

    lax.fori_loop(0, nch, chunk, 0)

    @pl.when(wid < 31)
    def _():
        pltpu.sync_copy(acc.at[pl.ds(0, NT)],
                        out_hbm.at[pl.ds(pl.multiple_of(base_node, 8), NT)])

    @pl.when(wid == 31)
    def _():
        pltpu.sync_copy(acc.at[pl.ds(0, NTL)],
                        out_hbm.at[pl.ds(N_NODES - NTL, NTL)])


def _make_segsum():
    return functools.partial(
        pl.kernel, _segsum_body,
        out_type=jax.ShapeDtypeStruct((N_NODES, LAT), jnp.float32),
        mesh=_mesh(),
        scratch_types=[
            pltpu.VMEM((CH,), jnp.int32),
            pltpu.VMEM((CH,), jnp.int32),
            pltpu.VMEM((16,), jnp.int32),
            pltpu.VMEM((CH, LAT), jnp.float32),
            pltpu.VMEM((ACC_R, LAT), jnp.float32),
            pltpu.SemaphoreType.DMA,
        ],
    )()


def _mm_body(relu, x_ref, w_ref, b_ref, o_ref):
    y = jnp.dot(x_ref[...], w_ref[...], preferred_element_type=jnp.float32)
    y = y + b_ref[...]
    if relu:
        y = jnp.maximum(y, 0.0)
    o_ref[...] = y


def _mm(x, w, b, relu=False, bn=2000):
    n, k = x.shape
    d = w.shape[1]
    return pl.pallas_call(
        functools.partial(_mm_body, relu),
        grid=(n // bn,),
        in_specs=[pl.BlockSpec((bn, k), lambda i: (i, 0)),
                  pl.BlockSpec((k, d), lambda i: (0, 0)),
                  pl.BlockSpec((1, d), lambda i: (0, 0))],
        out_specs=pl.BlockSpec((bn, d), lambda i: (i, 0)),
        out_shape=jax.ShapeDtypeStruct((n, d), jnp.float32),
    )(x, w, b.reshape(1, d))


def _mm2_body(x_ref, wa_ref, wb_ref, a_ref, b_ref):
    x = x_ref[...]
    a_ref[...] = jnp.dot(x, wa_ref[...], preferred_element_type=jnp.float32)
    b_ref[...] = jnp.dot(x, wb_ref[...], preferred_element_type=jnp.float32)


def _mm2(x, wa, wb, bn=2000):
    n, k = x.shape
    d = wa.shape[1]
    return pl.pallas_call(
        _mm2_body,
        grid=(n // bn,),
        in_specs=[pl.BlockSpec((bn, k), lambda i: (i, 0)),
                  pl.BlockSpec((k, d), lambda i: (0, 0)),
                  pl.BlockSpec((k, d), lambda i: (0, 0))],
        out_specs=[pl.BlockSpec((bn, d), lambda i: (i, 0)),
                   pl.BlockSpec((bn, d), lambda i: (i, 0))],
        out_shape=[jax.ShapeDtypeStruct((n, d), jnp.float32),
                   jax.ShapeDtypeStruct((n, d), jnp.float32)],
    )(x, wa, wb)


def _fuse_body(x_ref, w_ref, c_ref, a_ref, b_ref, o_ref):
    y = jnp.dot(x_ref[...], w_ref[...], preferred_element_type=jnp.float32)
    o_ref[...] = jnp.maximum(y + c_ref[...] + a_ref[...] + b_ref[...], 0.0)


def _edge_fuse(x, w, c, ag, bg, bn=2000):
    n, k = x.shape
    d = w.shape[1]
    return pl.pallas_call(
        _fuse_body,
        grid=(n // bn,),
        in_specs=[pl.BlockSpec((bn, k), lambda i: (i, 0)),
                  pl.BlockSpec((k, d), lambda i: (0, 0)),
                  pl.BlockSpec((1, d), lambda i: (0, 0)),
                  pl.BlockSpec((bn, d), lambda i: (i, 0)),
                  pl.BlockSpec((bn, d), lambda i: (i, 0))],
        out_specs=pl.BlockSpec((bn, d), lambda i: (i, 0)),
        out_shape=jax.ShapeDtypeStruct((n, d), jnp.float32),
    )(x, w, c, ag, bg)


def _fold_body(we_ref, be_ref, w1e0_ref, b10_ref, w20_ref, b20_ref,
               w1e1_ref, b11_ref, wf0_ref, cv0_ref, u_ref, cc1_ref):
    we = we_ref[...]
    w1e0 = w1e0_ref[...]
    wf0_ref[...] = jnp.dot(we, w1e0, preferred_element_type=jnp.float32)
    cv0_ref[...] = jnp.dot(be_ref[...], w1e0,
                           preferred_element_type=jnp.float32) + b10_ref[...]
    w1e1 = w1e1_ref[...]
    u_ref[...] = jnp.dot(w20_ref[...], w1e1,
                         preferred_element_type=jnp.float32)
    cc1_ref[...] = jnp.dot(b20_ref[...], w1e1,
                           preferred_element_type=jnp.float32) + b11_ref[...]


def _fold_params(we, be, w1e0, b10, w20, b20, w1e1, b11):
    return pl.pallas_call(
        _fold_body,
        out_shape=[jax.ShapeDtypeStruct((16, LAT), jnp.float32),
                   jax.ShapeDtypeStruct((1, LAT), jnp.float32),
                   jax.ShapeDtypeStruct((LAT, LAT), jnp.float32),
                   jax.ShapeDtypeStruct((1, LAT), jnp.float32)],
    )(we, be.reshape(1, LAT), w1e0, b10.reshape(1, LAT), w20,
      b20.reshape(1, LAT), w1e1, b11.reshape(1, LAT))


def _node_body(nodes_ref, s_ref, r_ref, cs_ref, cr_ref, g_ref,
               w2_ref, b2_ref, n1_ref, n1g_ref, c1_ref, n2_ref, c2_ref,
               out_ref, na_ref, sh_ref):
    i = pl.program_id(0)
    sblk = s_ref[...]
    sent = jnp.dot(sblk, w2_ref[...], preferred_element_type=jnp.float32)
    sent = sent + cs_ref[...] * b2_ref[...]
    recv = jnp.dot(r_ref[...], w2_ref[...], preferred_element_type=jnp.float32)
    recv = recv + cr_ref[...] * b2_ref[...]
    n1 = n1_ref[...]
    h = (jnp.dot(nodes_ref[...], n1[0:LAT], preferred_element_type=jnp.float32)
         + jnp.dot(sent, n1[LAT:2 * LAT], preferred_element_type=jnp.float32)
         + jnp.dot(recv, n1[2 * LAT:3 * LAT],
                   preferred_element_type=jnp.float32)
         + jnp.dot(g_ref[...], n1g_ref[...],
                   preferred_element_type=jnp.float32)
         + c1_ref[...])
    h = jnp.maximum(h, 0.0)
    new = jnp.dot(h, n2_ref[...], preferred_element_type=jnp.float32)
    new = new + c2_ref[...]
    out_ref[...] = new

    @pl.when(i == 0)
    def _():
        na_ref[...] = jnp.zeros_like(na_ref)
        sh_ref[...] = jnp.zeros_like(sh_ref)

    na_ref[...] += jnp.sum(new, axis=0, keepdims=True)
    sh_ref[...] += jnp.sum(sblk, axis=0, keepdims=True)


def _node_update(nodes_c, s, r, cs, cr, g, w2, b2, n1, c1, n2, c2, bn=2000):
    n = nodes_c.shape[0]
    zero = lambda i: (0, 0)
    return pl.pallas_call(
        _node_body,
        grid=(n // bn,),
        in_specs=[pl.BlockSpec((bn, LAT), lambda i: (i, 0)),
                  pl.BlockSpec((bn, LAT), lambda i: (i, 0)),
                  pl.BlockSpec((bn, LAT), lambda i: (i, 0)),
                  pl.BlockSpec((bn, 1), lambda i: (i, 0)),
                  pl.BlockSpec((bn, 1), lambda i: (i, 0)),
                  pl.BlockSpec((1, NOUT), zero),
                  pl.BlockSpec((LAT, LAT), zero),
                  pl.BlockSpec((1, LAT), zero),
                  pl.BlockSpec((3 * LAT, LAT), zero),
                  pl.BlockSpec((NOUT, LAT), zero),
                  pl.BlockSpec((1, LAT), zero),
                  pl.BlockSpec((LAT, LAT), zero),
                  pl.BlockSpec((1, LAT), zero)],
        out_specs=[pl.BlockSpec((bn, LAT), lambda i: (i, 0)),
                   pl.BlockSpec((1, LAT), zero),
                   pl.BlockSpec((1, LAT), zero)],
        out_shape=[jax.ShapeDtypeStruct((n, LAT), jnp.float32),
                   jax.ShapeDtypeStruct((1, LAT), jnp.float32),
                   jax.ShapeDtypeStruct((1, LAT), jnp.float32)],
    )(nodes_c, s, r, cs, cr, g, w2, b2.reshape(1, LAT), n1[:3 * LAT],
      n1[3 * LAT:], c1.reshape(1, LAT), n2, c2.reshape(1, LAT))


def _glob_body(na_ref, sh_ref, g_ref, w2_ref, b2_ref, g1_ref, d1_ref,
               g2_ref, d2_ref, g3_ref, d3_ref, wgn_ref, ccn_ref,
               go_ref, cv_ref):
    ea = jnp.dot(sh_ref[...], w2_ref[...],
                 preferred_element_type=jnp.float32) + N_EDGES * b2_ref[...]
    g1 = g1_ref[...]
    x = (jnp.dot(na_ref[...], g1[0:LAT], preferred_element_type=jnp.float32)
         + jnp.dot(ea, g1[LAT:2 * LAT], preferred_element_type=jnp.float32)
         + jnp.dot(g_ref[...], g1[2 * LAT:],
                   preferred_element_type=jnp.float32)
         + d1_ref[...])
    x = jnp.maximum(x, 0.0)
    x = jnp.dot(x, g2_ref[...], preferred_element_type=jnp.float32)
    x = jnp.maximum(x + d2_ref[...], 0.0)
    gn = jnp.dot(x, g3_ref[...], preferred_element_type=jnp.float32)
    gn = gn + d3_ref[...]
    go_ref[...] = gn
    cv_ref[...] = jnp.dot(gn, wgn_ref[...],
                          preferred_element_type=jnp.float32) + ccn_ref[...]


def _glob_mlp(na, sh, g, w2, b2, g1, d1, g2, d2, g3, d3, wgn, ccn):
    return pl.pallas_call(
        _glob_body,
        out_shape=[jax.ShapeDtypeStruct((1, NOUT), jnp.float32),
                   jax.ShapeDtypeStruct((1, LAT), jnp.float32)],
    )(na, sh, g, w2, b2.reshape(1, LAT), g1, d1.reshape(1, LAT), g2,
      d2.reshape(1, LAT), g3, d3.reshape(1, NOUT), wgn, ccn)


def kernel(nodes, edges, senders, receivers, params):
    senders = senders.astype(jnp.int32)
    receivers = receivers.astype(jnp.int32)
    wn, bn_ = params['embed_node']
    we, be = params['embed_edge']
    st0, st1 = params['steps']
    (w1_0, b1_0), (w2_0, b2_0) = st0['edge']
    (w1_1, b1_1), (w2_1, b2_1) = st1['edge']

    wf0, cvec0, u_mat, cconst1 = _fold_params(
        we, be, w1_0[0:LAT], b1_0, w2_0, b2_0, w1_1[0:LAT], b1_1)

    nodes1 = _mm(nodes, wn, bn_)
    a0, b0 = _mm2(nodes1, w1_0[LAT:2 * LAT], w1_0[2 * LAT:3 * LAT])
    ag0, bg0 = _gather2(a0, b0, senders, receivers)
    h1 = _edge_fuse(edges, wf0, cvec0, ag0, bg0)

    eids = jnp.arange(N_EDGES, dtype=jnp.int32)
    grid_n = jnp.arange(N_NODES + 1, dtype=jnp.int32)
    starts = jnp.arange(32, dtype=jnp.int32) * NT
    ends = jnp.concatenate([starts[1:], jnp.array([N_NODES], jnp.int32)])
    lane = jnp.arange(32, dtype=jnp.int32) * 8

    def _views(idx):
        sid, perm = lax.sort_key_val(idx, eids)
        spl = jnp.searchsorted(sid, grid_n).astype(jnp.int32)
        cnt = (spl[1:] - spl[:-1]).astype(jnp.float32).reshape(N_NODES, 1)
        tb = jnp.zeros((264,), jnp.int32)
        tb = tb.at[lane].set(spl[starts]).at[lane + 1].set(spl[ends])
        return sid, perm, cnt, tb

    sid_s, perm_s, cnt_s, tb_s = _views(senders)
    sid_r, perm_r, cnt_r, tb_r = _views(receivers)

    segsum = _make_segsum()
    s1 = segsum(h1, perm_s, sid_s, tb_s)
    r1 = segsum(h1, perm_r, sid_r, tb_r)

    g0 = jnp.zeros((1, NOUT), jnp.float32)
    (n1_0, c1_0), (n2_0, c2_0) = st0['node']
    nodes2, na1, sh1 = _node_update(
        nodes1, s1, r1, cnt_s, cnt_r, g0, w2_0, b2_0, n1_0, c1_0, n2_0, c2_0)

    (g1_0, d1_0), (g2_0, d2_0), (g3_0, d3_0) = st0['global']
    g1, cvec1 = _glob_mlp(na1, sh1, g0, w2_0, b2_0, g1_0, d1_0, g2_0, d2_0,
                          g3_0, d3_0, w1_1[3 * LAT:], cconst1)

    a1, b1 = _mm2(nodes2, w1_1[LAT:2 * LAT], w1_1[2 * LAT:3 * LAT])
    ag1, bg1 = _gather2(a1, b1, senders, receivers)
    h2 = _edge_fuse(h1, u_mat, cvec1, ag1, bg1)
    s2 = segsum(h2, perm_s, sid_s, tb_s)
    r2 = segsum(h2, perm_r, sid_r, tb_r)

    (n1_1, c1_1), (n2_1, c2_1) = st1['node']
    nodes3, na2, sh2 = _node_update(
        nodes2, s2, r2, cnt_s, cnt_r, g1, w2_1, b2_1, n1_1, c1_1, n2_1, c2_1)

    (g1_1, d1_1), (g2_1, d2_1), (g3_1, d3_1) = st1['global']
    zf = jnp.zeros((NOUT, LAT), jnp.float32)
    zc = jnp.zeros((1, LAT), jnp.float32)
    out, _ = _glob_mlp(na2, sh2, g1, w2_1, b2_1, g1_1, d1_1, g2_1, d2_1,
                       g3_1, d3_1, zf, zc)
    return out


# speedup vs baseline: 1.0010x; 1.0010x over previous
import functools

import jax
import jax.numpy as jnp
from jax import lax
from jax.experimental import pallas as pl
from jax.experimental.pallas import tpu as pltpu
from jax.experimental.pallas import tpu_sc as plsc

N_NODES = 10000
N_EDGES = 160000
LAT = 256
NOUT = 128
CH = 128
NCHUNK = N_EDGES // CH
HALF = N_NODES // 2
ACC_ROWS = HALF + 8


def _mesh():
    return plsc.VectorSubcoreMesh(core_axis_name="c", subcore_axis_name="s")


def _gather2_body(a_hbm, b_hbm, s_hbm, r_hbm, ag_hbm, bg_hbm,
                  sidx, ridx, abuf, bbuf, sem):
    cid = lax.axis_index("c")
    sid = lax.axis_index("s")
    wid = sid * 2 + cid
    nch = (NCHUNK - wid + 31) // 32

    def chunk(i, _):
        base = pl.multiple_of((wid + i * 32) * CH, 8)
        pltpu.sync_copy(s_hbm.at[pl.ds(base, CH)], sidx)
        pltpu.sync_copy(r_hbm.at[pl.ds(base, CH)], ridx)
        ca = pltpu.async_copy(a_hbm.at[sidx], abuf, sem)
        cb = pltpu.async_copy(b_hbm.at[ridx], bbuf, sem)
        ca.wait()
        cb.wait()
        pltpu.sync_copy(abuf, ag_hbm.at[pl.ds(base, CH)])
        pltpu.sync_copy(bbuf, bg_hbm.at[pl.ds(base, CH)])
        return 0

    lax.fori_loop(0, nch, chunk, 0)


def _gather2(a, b, senders, receivers):
    f = functools.partial(
        pl.kernel, _gather2_body,
        out_type=(jax.ShapeDtypeStruct((N_EDGES, LAT), jnp.float32),
                  jax.ShapeDtypeStruct((N_EDGES, LAT), jnp.float32)),
        mesh=_mesh(),
        scratch_types=[
            pltpu.VMEM((CH,), jnp.int32),
            pltpu.VMEM((CH,), jnp.int32),
            pltpu.VMEM((CH, LAT), jnp.float32),
            pltpu.VMEM((CH, LAT), jnp.float32),
            pltpu.SemaphoreType.DMA,
        ],
    )()
    return f(a, b, senders, receivers)


NT2 = 312
NTL2 = 328
DUMP = 328
ACC_R = 336


def _segsum_body(h_hbm, perm_hbm, nid_hbm, tb_hbm, out_hbm,
                 pidx, nbuf, lidx, tbuf, hbuf, acc, sem):
    cid = lax.axis_index("c")
    sid = lax.axis_index("s")
    wid = sid * 2 + cid
    base_node = NT2 * wid
    nn = jnp.where(wid == 31, NTL2, NT2)

    def zrow(r, _):
        for k in range(LAT // 16):
            acc[r, pl.ds(k * 16, 16)] = jnp.zeros((16,), jnp.float32)
        return 0

    lax.fori_loop(0, ACC_R, zrow, 0)

    pltpu.sync_copy(tb_hbm.at[pl.ds(pl.multiple_of(8 * wid, 8), 16)], tbuf)
    tv = tbuf[...]
    lo = tv[0]
    hi = tv[1]
    ca0 = (lo // CH) * CH
    nch = (hi - ca0 + CH - 1) // CH

    def chunk(i, _):
        ca = pl.multiple_of(ca0 + i * CH, 8)
        pltpu.sync_copy(perm_hbm.at[pl.ds(ca, CH)], pidx)
        pltpu.sync_copy(nid_hbm.at[pl.ds(ca, CH)], nbuf)
        cpy = pltpu.async_copy(h_hbm.at[pidx], hbuf, sem)
        for g in range(CH // 16):
            nv = nbuf[pl.ds(g * 16, 16)] - base_node
            ok = jnp.logical_and(nv >= 0, nv < nn)
            lidx[pl.ds(g * 16, 16)] = jnp.where(ok, nv, DUMP)
        cpy.wait()
        for g in range(CH // 16):
            lv = lidx[pl.ds(g * 16, 16)]
            for j in range(16):
                r = g * 16 + j
                li = lv[j]
                for k in range(LAT // 16):
                    sl = pl.ds(k * 16, 16)
                    plsc.addupdate(acc.at[li, sl], hbuf[r, sl])
        return 0

    lax.fori_loop(0, nch, chunk, 0)

    @pl.when(wid < 31)
    def _():
        pltpu.sync_copy(acc.at[pl.ds(0, NT2)],
                        out_hbm.at[pl.ds(pl.multiple_of(base_node, 8), NT2)])

    @pl.when(wid == 31)
    def _():
        pltpu.sync_copy(acc.at[pl.ds(0, NTL2)],
                        out_hbm.at[pl.ds(N_NODES - NTL2, NTL2)])


def _make_segsum():
    return functools.partial(
        pl.kernel, _segsum_body,
        out_type=jax.ShapeDtypeStruct((N_NODES, LAT), jnp.float32),
        mesh=_mesh(),
        scratch_types=[
            pltpu.VMEM((CH,), jnp.int32),
            pltpu.VMEM((CH,), jnp.int32),
            pltpu.VMEM((CH,), jnp.int32),
            pltpu.VMEM((16,), jnp.int32),
            pltpu.VMEM((CH, LAT), jnp.float32),
            pltpu.VMEM((ACC_R, LAT), jnp.float32),
            pltpu.SemaphoreType.DMA,
        ],
    )()


def _mm_body(relu, x_ref, w_ref, b_ref, o_ref):
    y = jnp.dot(x_ref[...], w_ref[...], preferred_element_type=jnp.float32)
    y = y + b_ref[...]
    if relu:
        y = jnp.maximum(y, 0.0)
    o_ref[...] = y


def _mm(x, w, b, relu=False, bn=2000):
    n, k = x.shape
    d = w.shape[1]
    return pl.pallas_call(
        functools.partial(_mm_body, relu),
        grid=(n // bn,),
        in_specs=[pl.BlockSpec((bn, k), lambda i: (i, 0)),
                  pl.BlockSpec((k, d), lambda i: (0, 0)),
                  pl.BlockSpec((1, d), lambda i: (0, 0))],
        out_specs=pl.BlockSpec((bn, d), lambda i: (i, 0)),
        out_shape=jax.ShapeDtypeStruct((n, d), jnp.float32),
    )(x, w, b.reshape(1, d))


def _mm2_body(x_ref, wa_ref, wb_ref, a_ref, b_ref):
    x = x_ref[...]
    a_ref[...] = jnp.dot(x, wa_ref[...], preferred_element_type=jnp.float32)
    b_ref[...] = jnp.dot(x, wb_ref[...], preferred_element_type=jnp.float32)


def _mm2(x, wa, wb, bn=2000):
    n, k = x.shape
    d = wa.shape[1]
    return pl.pallas_call(
        _mm2_body,
        grid=(n // bn,),
        in_specs=[pl.BlockSpec((bn, k), lambda i: (i, 0)),
                  pl.BlockSpec((k, d), lambda i: (0, 0)),
                  pl.BlockSpec((k, d), lambda i: (0, 0))],
        out_specs=[pl.BlockSpec((bn, d), lambda i: (i, 0)),
                   pl.BlockSpec((bn, d), lambda i: (i, 0))],
        out_shape=[jax.ShapeDtypeStruct((n, d), jnp.float32),
                   jax.ShapeDtypeStruct((n, d), jnp.float32)],
    )(x, wa, wb)


def _fuse_body(x_ref, w_ref, c_ref, a_ref, b_ref, o_ref):
    y = jnp.dot(x_ref[...], w_ref[...], preferred_element_type=jnp.float32)
    o_ref[...] = jnp.maximum(y + c_ref[...] + a_ref[...] + b_ref[...], 0.0)


def _edge_fuse(x, w, c, ag, bg, bn=2000):
    n, k = x.shape
    d = w.shape[1]
    return pl.pallas_call(
        _fuse_body,
        grid=(n // bn,),
        in_specs=[pl.BlockSpec((bn, k), lambda i: (i, 0)),
                  pl.BlockSpec((k, d), lambda i: (0, 0)),
                  pl.BlockSpec((1, d), lambda i: (0, 0)),
                  pl.BlockSpec((bn, d), lambda i: (i, 0)),
                  pl.BlockSpec((bn, d), lambda i: (i, 0))],
        out_specs=pl.BlockSpec((bn, d), lambda i: (i, 0)),
        out_shape=jax.ShapeDtypeStruct((n, d), jnp.float32),
    )(x, w, c, ag, bg)


def _fold_body(we_ref, be_ref, w1e0_ref, b10_ref, w20_ref, b20_ref,
               w1e1_ref, b11_ref, wf0_ref, cv0_ref, u_ref, cc1_ref):
    we = we_ref[...]
    w1e0 = w1e0_ref[...]
    wf0_ref[...] = jnp.dot(we, w1e0, preferred_element_type=jnp.float32)
    cv0_ref[...] = jnp.dot(be_ref[...], w1e0,
                           preferred_element_type=jnp.float32) + b10_ref[...]
    w1e1 = w1e1_ref[...]
    u_ref[...] = jnp.dot(w20_ref[...], w1e1,
                         preferred_element_type=jnp.float32)
    cc1_ref[...] = jnp.dot(b20_ref[...], w1e1,
                           preferred_element_type=jnp.float32) + b11_ref[...]


def _fold_params(we, be, w1e0, b10, w20, b20, w1e1, b11):
    return pl.pallas_call(
        _fold_body,
        out_shape=[jax.ShapeDtypeStruct((16, LAT), jnp.float32),
                   jax.ShapeDtypeStruct((1, LAT), jnp.float32),
                   jax.ShapeDtypeStruct((LAT, LAT), jnp.float32),
                   jax.ShapeDtypeStruct((1, LAT), jnp.float32)],
    )(we, be.reshape(1, LAT), w1e0, b10.reshape(1, LAT), w20,
      b20.reshape(1, LAT), w1e1, b11.reshape(1, LAT))


def _node_body(nodes_ref, s_ref, r_ref, cs_ref, cr_ref, g_ref,
               w2_ref, b2_ref, n1_ref, n1g_ref, c1_ref, n2_ref, c2_ref,
               out_ref, na_ref, sh_ref):
    i = pl.program_id(0)
    sblk = s_ref[...]
    sent = jnp.dot(sblk, w2_ref[...], preferred_element_type=jnp.float32)
    sent = sent + cs_ref[...] * b2_ref[...]
    recv = jnp.dot(r_ref[...], w2_ref[...], preferred_element_type=jnp.float32)
    recv = recv + cr_ref[...] * b2_ref[...]
    n1 = n1_ref[...]
    h = (jnp.dot(nodes_ref[...], n1[0:LAT], preferred_element_type=jnp.float32)
         + jnp.dot(sent, n1[LAT:2 * LAT], preferred_element_type=jnp.float32)
         + jnp.dot(recv, n1[2 * LAT:3 * LAT],
                   preferred_element_type=jnp.float32)
         + jnp.dot(g_ref[...], n1g_ref[...],
                   preferred_element_type=jnp.float32)
         + c1_ref[...])
    h = jnp.maximum(h, 0.0)
    new = jnp.dot(h, n2_ref[...], preferred_element_type=jnp.float32)
    new = new + c2_ref[...]
    out_ref[...] = new

    @pl.when(i == 0)
    def _():
        na_ref[...] = jnp.zeros_like(na_ref)
        sh_ref[...] = jnp.zeros_like(sh_ref)

    na_ref[...] += jnp.sum(new, axis=0, keepdims=True)
    sh_ref[...] += jnp.sum(sblk, axis=0, keepdims=True)


def _node_update(nodes_c, s, r, cs, cr, g, w2, b2, n1, c1, n2, c2, bn=2000):
    n = nodes_c.shape[0]
    zero = lambda i: (0, 0)
    return pl.pallas_call(
        _node_body,
        grid=(n // bn,),
        in_specs=[pl.BlockSpec((bn, LAT), lambda i: (i, 0)),
                  pl.BlockSpec((bn, LAT), lambda i: (i, 0)),
                  pl.BlockSpec((bn, LAT), lambda i: (i, 0)),
                  pl.BlockSpec((bn, 1), lambda i: (i, 0)),
                  pl.BlockSpec((bn, 1), lambda i: (i, 0)),
                  pl.BlockSpec((1, NOUT), zero),
                  pl.BlockSpec((LAT, LAT), zero),
                  pl.BlockSpec((1, LAT), zero),
                  pl.BlockSpec((3 * LAT, LAT), zero),
                  pl.BlockSpec((NOUT, LAT), zero),
                  pl.BlockSpec((1, LAT), zero),
                  pl.BlockSpec((LAT, LAT), zero),
                  pl.BlockSpec((1, LAT), zero)],
        out_specs=[pl.BlockSpec((bn, LAT), lambda i: (i, 0)),
                   pl.BlockSpec((1, LAT), zero),
                   pl.BlockSpec((1, LAT), zero)],
        out_shape=[jax.ShapeDtypeStruct((n, LAT), jnp.float32),
                   jax.ShapeDtypeStruct((1, LAT), jnp.float32),
                   jax.ShapeDtypeStruct((1, LAT), jnp.float32)],
    )(nodes_c, s, r, cs, cr, g, w2, b2.reshape(1, LAT), n1[:3 * LAT],
      n1[3 * LAT:], c1.reshape(1, LAT), n2, c2.reshape(1, LAT))


def _glob_body(na_ref, sh_ref, g_ref, w2_ref, b2_ref, g1_ref, d1_ref,
               g2_ref, d2_ref, g3_ref, d3_ref, wgn_ref, ccn_ref,
               go_ref, cv_ref):
    ea = jnp.dot(sh_ref[...], w2_ref[...],
                 preferred_element_type=jnp.float32) + N_EDGES * b2_ref[...]
    g1 = g1_ref[...]
    x = (jnp.dot(na_ref[...], g1[0:LAT], preferred_element_type=jnp.float32)
         + jnp.dot(ea, g1[LAT:2 * LAT], preferred_element_type=jnp.float32)
         + jnp.dot(g_ref[...], g1[2 * LAT:],
                   preferred_element_type=jnp.float32)
         + d1_ref[...])
    x = jnp.maximum(x, 0.0)
    x = jnp.dot(x, g2_ref[...], preferred_element_type=jnp.float32)
    x = jnp.maximum(x + d2_ref[...], 0.0)
    gn = jnp.dot(x, g3_ref[...], preferred_element_type=jnp.float32)
    gn = gn + d3_ref[...]
    go_ref[...] = gn
    cv_ref[...] = jnp.dot(gn, wgn_ref[...],
                          preferred_element_type=jnp.float32) + ccn_ref[...]


def _glob_mlp(na, sh, g, w2, b2, g1, d1, g2, d2, g3, d3, wgn, ccn):
    return pl.pallas_call(
        _glob_body,
        out_shape=[jax.ShapeDtypeStruct((1, NOUT), jnp.float32),
                   jax.ShapeDtypeStruct((1, LAT), jnp.float32)],
    )(na, sh, g, w2, b2.reshape(1, LAT), g1, d1.reshape(1, LAT), g2,
      d2.reshape(1, LAT), g3, d3.reshape(1, NOUT), wgn, ccn)


def kernel(nodes, edges, senders, receivers, params):
    senders = senders.astype(jnp.int32)
    receivers = receivers.astype(jnp.int32)
    wn, bn_ = params['embed_node']
    we, be = params['embed_edge']
    st0, st1 = params['steps']
    (w1_0, b1_0), (w2_0, b2_0) = st0['edge']
    (w1_1, b1_1), (w2_1, b2_1) = st1['edge']

    wf0, cvec0, u_mat, cconst1 = _fold_params(
        we, be, w1_0[0:LAT], b1_0, w2_0, b2_0, w1_1[0:LAT], b1_1)

    nodes1 = _mm(nodes, wn, bn_)
    a0, b0 = _mm2(nodes1, w1_0[LAT:2 * LAT], w1_0[2 * LAT:3 * LAT])
    ag0, bg0 = _gather2(a0, b0, senders, receivers)
    h1 = _edge_fuse(edges, wf0, cvec0, ag0, bg0)

    eids = jnp.arange(N_EDGES, dtype=jnp.int32)
    grid_n = jnp.arange(N_NODES + 1, dtype=jnp.int32)
    starts = jnp.arange(32, dtype=jnp.int32) * NT2
    ends = jnp.concatenate([starts[1:], jnp.array([N_NODES], jnp.int32)])
    lane = jnp.arange(32, dtype=jnp.int32) * 8

    def _views(idx):
        sid, perm = lax.sort_key_val(idx, eids)
        spl = jnp.searchsorted(sid, grid_n).astype(jnp.int32)
        cnt = (spl[1:] - spl[:-1]).astype(jnp.float32).reshape(N_NODES, 1)
        tb = jnp.zeros((264,), jnp.int32)
        tb = tb.at[lane].set(spl[starts]).at[lane + 1].set(spl[ends])
        return sid, perm, cnt, tb

    sid_s, perm_s, cnt_s, tb_s = _views(senders)
    sid_r, perm_r, cnt_r, tb_r = _views(receivers)

    segsum = _make_segsum()
    s1 = segsum(h1, perm_s, sid_s, tb_s)
    r1 = segsum(h1, perm_r, sid_r, tb_r)

    g0 = jnp.zeros((1, NOUT), jnp.float32)
    (n1_0, c1_0), (n2_0, c2_0) = st0['node']
    nodes2, na1, sh1 = _node_update(
        nodes1, s1, r1, cnt_s, cnt_r, g0, w2_0, b2_0, n1_0, c1_0, n2_0, c2_0)

    (g1_0, d1_0), (g2_0, d2_0), (g3_0, d3_0) = st0['global']
    g1, cvec1 = _glob_mlp(na1, sh1, g0, w2_0, b2_0, g1_0, d1_0, g2_0, d2_0,
                          g3_0, d3_0, w1_1[3 * LAT:], cconst1)

    a1, b1 = _mm2(nodes2, w1_1[LAT:2 * LAT], w1_1[2 * LAT:3 * LAT])
    ag1, bg1 = _gather2(a1, b1, senders, receivers)
    h2 = _edge_fuse(h1, u_mat, cvec1, ag1, bg1)
    s2 = segsum(h2, perm_s, sid_s, tb_s)
    r2 = segsum(h2, perm_r, sid_r, tb_r)

    (n1_1, c1_1), (n2_1, c2_1) = st1['node']
    nodes3, na2, sh2 = _node_update(
        nodes2, s2, r2, cnt_s, cnt_r, g1, w2_1, b2_1, n1_1, c1_1, n2_1, c2_1)

    (g1_1, d1_1), (g2_1, d2_1), (g3_1, d3_1) = st1['global']
    zf = jnp.zeros((NOUT, LAT), jnp.float32)
    zc = jnp.zeros((1, LAT), jnp.float32)
    out, _ = _glob_mlp(na2, sh2, g1, w2_1, b2_1, g1_1, d1_1, g2_1, d2_1,
                       g3_1, d3_1, zf, zc)
    return out
